# baseline (speedup 1.0000x reference)
import functools

import jax
import jax.numpy as jnp
from jax import lax
from jax.experimental import pallas as pl
from jax.experimental.pallas import tpu as pltpu
from jax.experimental.pallas import tpu_sc as plsc

_BATCH = 16384
_D = 64
_NC = 2
_NS = 16
_NW = _NC * _NS
_BPW = _BATCH // _NW
_CHUNK = 128
_NCHUNK = _BPW // _CHUNK
_LANES = 16

_mesh = plsc.VectorSubcoreMesh(core_axis_name="c", subcore_axis_name="s")


@functools.partial(
    pl.kernel,
    out_type=jax.ShapeDtypeStruct((_NW, _LANES), jnp.float32),
    mesh=_mesh,
    scratch_types=[
        pltpu.VMEM((_NCHUNK, _CHUNK), jnp.int32),
        pltpu.VMEM((_BPW, _D), jnp.float32),
        pltpu.VMEM((_BPW, _D), jnp.float32),
        pltpu.VMEM((_LANES,), jnp.float32),
        pltpu.SemaphoreType.DMA,
    ],
    compiler_params=pltpu.CompilerParams(use_tc_tiling_on_sc=False),
)
def _center_loss_partials(feat_hbm, lab_hbm, cent_hbm, out_hbm,
                          idx_v, cent_v, feat_v, acc_v, sem):
    wid = lax.axis_index("s") * _NC + lax.axis_index("c")
    base = wid * _BPW

    pltpu.sync_copy(lab_hbm.at[wid], idx_v)
    gathers = [
        pltpu.async_copy(cent_hbm.at[idx_v.at[k]],
                         cent_v.at[pl.ds(k * _CHUNK, _CHUNK)], sem)
        for k in range(_NCHUNK)
    ]
    pltpu.sync_copy(feat_hbm.at[pl.ds(base, _BPW)], feat_v)
    for g in gathers:
        g.wait()

    ngrp = _D // _LANES

    def row(i, accs):
        out = []
        for j in range(ngrp):
            f = feat_v[i, pl.ds(j * _LANES, _LANES)]
            c = cent_v[i, pl.ds(j * _LANES, _LANES)]
            d = f - c
            out.append(accs[j] + d * d)
        return tuple(out)

    accs = lax.fori_loop(
        0, _BPW, row,
        tuple(jnp.zeros((_LANES,), jnp.float32) for _ in range(ngrp)))
    acc_v[...] = (accs[0] + accs[1]) + (accs[2] + accs[3])
    pltpu.sync_copy(acc_v, out_hbm.at[wid])


def kernel(features, labels, centers):
    labels = labels.astype(jnp.int32).reshape(_NW, _NCHUNK, _CHUNK)
    partials = _center_loss_partials(features, labels, centers)
    return jnp.sum(partials) / jnp.float32(_BATCH)
